# baseline (speedup 1.0000x reference)
import functools

import jax
import jax.numpy as jnp
from jax import lax
from jax.experimental import pallas as pl
from jax.experimental.pallas import tpu as pltpu
from jax.experimental.pallas import tpu_sc as plsc

DIM = 64
LANES = 16
NC, NS = 2, 16
NW = NC * NS
C = 128

def _rsqrt(x):
    i = plsc.bitcast(x, jnp.int32)
    i = jnp.full((LANES,), 0x5F3759DF, jnp.int32) - lax.shift_right_logical(i, 1)
    y = plsc.bitcast(i, jnp.float32)
    for _ in range(3):
        y = y * (1.5 - 0.5 * x * y * y)
    return y


def _score_pass(h_rows, t_rows, r_rows, out_ref, out_base):
    def group_body(g, carry):
        rows = g * LANES + lax.iota(jnp.int32, LANES)
        zero = jnp.zeros((LANES,), jnp.float32)
        hh = zero; tt = zero; rr = zero
        hr = zero; ht = zero; rt = zero
        for d in range(DIM):
            col = jnp.full((LANES,), d, jnp.int32)
            hv = plsc.load_gather(h_rows, [rows, col])
            tv = plsc.load_gather(t_rows, [rows, col])
            rv = plsc.load_gather(r_rows, [rows, col])
            hh = hh + hv * hv
            tt = tt + tv * tv
            rr = rr + rv * rv
            hr = hr + hv * rv
            ht = ht + hv * tv
            rt = rt + rv * tv
        ih = _rsqrt(jnp.maximum(hh, 1e-24))
        it = _rsqrt(jnp.maximum(tt, 1e-24))
        g2 = ((hh * ih) * ih + rr + (tt * it) * it
              + 2.0 * ((hr * ih) - (ht * ih) * it - (rt * it)))
        g2 = jnp.maximum(g2, 0.0)
        res = g2 * _rsqrt(jnp.maximum(g2, 1e-24))
        out_ref[pl.ds(out_base + g * LANES, LANES)] = -res
        return carry

    lax.fori_loop(0, C // LANES, group_body, jnp.int32(0))


def _make_sc_call(batch):
    assert batch % NW == 0
    pw = batch // NW
    assert pw % C == 0
    n_chunks = pw // C
    mesh = plsc.VectorSubcoreMesh(core_axis_name="c", subcore_axis_name="s")

    @functools.partial(
        pl.kernel,
        mesh=mesh,
        compiler_params=pltpu.CompilerParams(
            use_tc_tiling_on_sc=False, needs_layout_passes=False),
        out_type=(
            jax.ShapeDtypeStruct((batch,), jnp.float32),
            jax.ShapeDtypeStruct((batch,), jnp.float32),
        ),
        scratch_types=[
            pltpu.VMEM((pw,), jnp.int32),
            pltpu.VMEM((pw,), jnp.int32),
            pltpu.VMEM((pw,), jnp.int32),
            pltpu.VMEM((pw,), jnp.int32),
            pltpu.VMEM((pw,), jnp.int32),
            pltpu.VMEM((C, DIM), jnp.float32),
            pltpu.VMEM((C, DIM), jnp.float32),
            pltpu.VMEM((C, DIM), jnp.float32),
            pltpu.VMEM((C, DIM), jnp.float32),
            pltpu.VMEM((C, DIM), jnp.float32),
            pltpu.VMEM((pw,), jnp.float32),
            pltpu.VMEM((pw,), jnp.float32),
            pltpu.SemaphoreType.DMA,
            pltpu.SemaphoreType.DMA,
        ],
    )
    def sc_call(heads, tails, nheads, ntails, rels, ent, rel_emb,
                out_g, out_n,
                hidx, tidx, nhidx, ntidx, ridx,
                h_rows, t_rows, r_rows, nh_rows, nt_rows,
                og, on, sem_a, sem_b):
        wid = lax.axis_index("s") * NC + lax.axis_index("c")
        base = pl.multiple_of(wid * pw, pw)
        pltpu.sync_copy(heads.at[pl.ds(base, pw)], hidx)
        pltpu.sync_copy(tails.at[pl.ds(base, pw)], tidx)
        pltpu.sync_copy(nheads.at[pl.ds(base, pw)], nhidx)
        pltpu.sync_copy(ntails.at[pl.ds(base, pw)], ntidx)
        pltpu.sync_copy(rels.at[pl.ds(base, pw)], ridx)

        def chunk_body(ch, carry):
            off = pl.multiple_of(ch * C, C)
            a1 = pltpu.async_copy(ent.at[hidx.at[pl.ds(off, C)]], h_rows, sem_a)
            a2 = pltpu.async_copy(ent.at[tidx.at[pl.ds(off, C)]], t_rows, sem_a)
            a3 = pltpu.async_copy(rel_emb.at[ridx.at[pl.ds(off, C)]], r_rows, sem_a)
            a4 = pltpu.async_copy(ent.at[nhidx.at[pl.ds(off, C)]], nh_rows, sem_b)
            a5 = pltpu.async_copy(ent.at[ntidx.at[pl.ds(off, C)]], nt_rows, sem_b)
            a1.wait()
            a2.wait()
            a3.wait()
            _score_pass(h_rows, t_rows, r_rows, og, off)
            a4.wait()
            a5.wait()
            _score_pass(nh_rows, nt_rows, r_rows, on, off)
            return carry

        lax.fori_loop(0, n_chunks, chunk_body, jnp.int32(0))
        pltpu.sync_copy(og, out_g.at[pl.ds(base, pw)])
        pltpu.sync_copy(on, out_n.at[pl.ds(base, pw)])

    return sc_call


def kernel(heads, tails, negative_heads, negative_tails, relations,
           entity_embeddings, relation_embeddings):
    batch = heads.shape[0]
    sc_call = _make_sc_call(batch)
    return sc_call(heads, tails, negative_heads, negative_tails, relations,
                   entity_embeddings, relation_embeddings)
